# baseline (speedup 1.0000x reference)
import functools

import jax
import jax.numpy as jnp
from jax import lax
from jax.experimental import pallas as pl
from jax.experimental.pallas import tpu as pltpu
from jax.experimental.pallas import tpu_sc as plsc

N_NODES = 10000
D = 128
D_EDGE = 16
GATE = 16
NUM_CANON = 5000

NC = 2
NS = 16
L = 16
NW = NC * NS

N_EDGES = 320000
EC = 80
KE = 125
E_PER_W = EC * KE
KB = 5
NG = KE // KB

NPAD = 10240
ROWS_PER_TILE = NPAD // NS
ROWS_PER_W = NPAD // NW
CPAD = 5120
CROWS_PER_TILE = CPAD // NS
C2 = 64
K2 = ROWS_PER_W // C2

_MESH = plsc.VectorSubcoreMesh(
    core_axis_name="c", subcore_axis_name="s", num_cores=NC, num_subcores=NS)


KB8 = 8
NB = KE // KB8
KTAIL = KE - NB * KB8


def _sc1_body(x_hbm, src_hbm, dst_hbm, a_hbm, out_hbm,
              src_v, dst8_v, dst5_v, xg0_v, xg1_v, a_v, acc,
              sem0, sem1):
    cid = lax.axis_index("c")
    sid = lax.axis_index("s")
    wid = sid * NC + cid
    xg = (xg0_v, xg1_v)
    gsems = (sem0, sem1)

    def zrow(r, _):
        for c in range(D // L):
            xg0_v[r, pl.ds(c * L, L)] = jnp.zeros((L,), jnp.float32)
        return jnp.int32(0)
    lax.fori_loop(jnp.int32(0), jnp.int32(EC), zrow, jnp.int32(0))
    r0 = sid * ROWS_PER_TILE
    for b in range(ROWS_PER_TILE // EC):
        pltpu.sync_copy(xg0_v, acc.at[pl.ds(r0 + b * EC, EC)])
    plsc.subcore_barrier()

    pltpu.sync_copy(src_hbm.at[wid], src_v)

    def mul(xg_v):
        def row(r, _):
            for k in range(D // (2 * L)):
                ab = plsc.bitcast(a_v[r, pl.ds(k * L, L)], jnp.bfloat16)
                a0, a1 = plsc.unpack(ab, format=plsc.PackFormat.INTERLEAVED,
                                     preferred_element_type=jnp.float32)
                s0 = pl.ds(k * 2 * L, L)
                s1 = pl.ds(k * 2 * L + L, L)
                xg_v[r, s0] = xg_v[r, s0] * a0
                xg_v[r, s1] = xg_v[r, s1] * a1
            return jnp.int32(0)
        lax.fori_loop(jnp.int32(0), jnp.int32(EC), row, jnp.int32(0))

    pltpu.async_copy(x_hbm.at[src_v.at[jnp.int32(0)]], xg0_v, sem0)

    def step(j, c, dst_ref, ci, fire_next):
        p, q = c % 2, (c + 1) % 2
        if fire_next:
            pltpu.async_copy(x_hbm.at[src_v.at[j + 1]], xg[q], gsems[q])
        pltpu.make_async_copy(x_hbm.at[src_v.at[j]], xg[p], gsems[p]).wait()
        pltpu.sync_copy(a_hbm.at[wid, j], a_v)
        mul(xg[p])
        xg_v2 = xg[p]
        pltpu.sync_copy(xg_v2, acc.at[dst_ref.at[jnp.int32(ci)]], add=True)

    def block(b, _):
        pltpu.sync_copy(dst_hbm.at[wid, pl.ds(b * KB8, KB8)], dst8_v)
        for c in range(KB8):
            step(b * KB8 + c, c, dst8_v, c, True)
        return jnp.int32(0)
    lax.fori_loop(jnp.int32(0), jnp.int32(NB), block, jnp.int32(0))

    pltpu.sync_copy(dst_hbm.at[wid, pl.ds(NB * KB8, KTAIL)], dst5_v)
    for c in range(KTAIL):
        step(jnp.int32(NB * KB8 + c), c, dst5_v, c, c + 1 < KTAIL)

    plsc.subcore_barrier()
    for b in range(ROWS_PER_TILE // EC):
        pltpu.sync_copy(acc.at[pl.ds(r0 + b * EC, EC)], xg0_v)
        pltpu.sync_copy(xg0_v, out_hbm.at[cid, pl.ds(r0 + b * EC, EC)])


_sc1 = functools.partial(
    pl.kernel,
    out_type=jax.ShapeDtypeStruct((NC, NPAD, D), jnp.float32),
    mesh=_MESH,
    scratch_types=[
        pltpu.VMEM((KE, EC), jnp.int32),
        pltpu.VMEM((KB8, EC), jnp.int32),
        pltpu.VMEM((KTAIL, EC), jnp.int32),
        pltpu.VMEM((EC, D), jnp.float32),
        pltpu.VMEM((EC, D), jnp.float32),
        pltpu.VMEM((EC, D // 2), jnp.int32),
        pltpu.VMEM_SHARED((NPAD, D), jnp.float32),
        pltpu.SemaphoreType.DMA,
        pltpu.SemaphoreType.DMA,
    ],
    compiler_params=pltpu.CompilerParams(needs_layout_passes=False),
)(_sc1_body)


def _sc2_body(xa_hbm, z_hbm, can_hbm, sums_out, z_v, c_v, seg_v, xa_v, acc):
    cid = lax.axis_index("c")
    sid = lax.axis_index("s")
    wid = sid * NC + cid

    def zrow(r, _):
        for c in range(D // L):
            xa_v[r, pl.ds(c * L, L)] = jnp.zeros((L,), jnp.float32)
        return jnp.int32(0)
    lax.fori_loop(jnp.int32(0), jnp.int32(C2), zrow, jnp.int32(0))
    r0 = sid * CROWS_PER_TILE
    for b in range(CROWS_PER_TILE // C2):
        pltpu.sync_copy(xa_v, acc.at[pl.ds(r0 + b * C2, C2)])
    plsc.subcore_barrier()

    pltpu.sync_copy(z_hbm.at[wid], z_v)
    pltpu.sync_copy(can_hbm.at[wid], c_v)
    base = wid * ROWS_PER_W
    for j in range(K2):
        jj = jnp.int32(j)
        for i in range(C2 // L):
            sl = pl.ds(i * L, L)
            zz = z_v[jj, sl]
            cc = c_v[jj, sl]
            dummy = NUM_CANON + lax.iota(jnp.int32, L) + (i * L)
            seg_v[jj, sl] = jnp.where(zz > 1, cc, dummy)
        pltpu.sync_copy(xa_hbm.at[pl.ds(base + j * C2, C2)], xa_v)
        pltpu.sync_copy(xa_v, acc.at[seg_v.at[jj]], add=True)

    plsc.subcore_barrier()
    for b in range(CROWS_PER_TILE // C2):
        pltpu.sync_copy(acc.at[pl.ds(r0 + b * C2, C2)], xa_v)
        pltpu.sync_copy(xa_v, sums_out.at[cid, pl.ds(r0 + b * C2, C2)])


_sc2 = functools.partial(
    pl.kernel,
    out_type=jax.ShapeDtypeStruct((NC, CPAD, D), jnp.float32),
    mesh=_MESH,
    scratch_types=[
        pltpu.VMEM((K2, C2), jnp.int32),
        pltpu.VMEM((K2, C2), jnp.int32),
        pltpu.VMEM((K2, C2), jnp.int32),
        pltpu.VMEM((C2, D), jnp.float32),
        pltpu.VMEM_SHARED((CPAD, D), jnp.float32),
    ],
)(_sc2_body)


def _sc3_body(t_hbm, can_hbm, out_hbm, c_v, g_v, sem):
    cid = lax.axis_index("c")
    sid = lax.axis_index("s")
    wid = sid * NC + cid
    pltpu.sync_copy(can_hbm.at[wid], c_v)
    base = wid * ROWS_PER_W
    for j in range(K2):
        jj = jnp.int32(j)
        pltpu.async_copy(t_hbm.at[c_v.at[jj]], g_v, sem).wait()
        pltpu.sync_copy(g_v, out_hbm.at[pl.ds(base + j * C2, C2)])


_sc3 = functools.partial(
    pl.kernel,
    out_type=jax.ShapeDtypeStruct((NPAD, D), jnp.float32),
    mesh=_MESH,
    scratch_types=[
        pltpu.VMEM((K2, C2), jnp.int32),
        pltpu.VMEM((C2, D), jnp.float32),
        pltpu.SemaphoreType.DMA,
    ],
)(_sc3_body)


BLK_E = 4000
BLK_N = 1024
BLK_C = 1024


def _tc0_body(ea_ref, wt_ref, o_ref):
    o_ref[...] = jnp.dot(ea_ref[...], wt_ref[...],
                         preferred_element_type=jnp.float32,
                         precision=lax.Precision.HIGHEST).astype(jnp.bfloat16)


_tc0 = pl.pallas_call(
    _tc0_body,
    grid=(N_EDGES // BLK_E,),
    in_specs=[pl.BlockSpec((BLK_E, D_EDGE), lambda i: (i, jnp.int32(0))),
              pl.BlockSpec((D_EDGE, D), lambda i: (jnp.int32(0), jnp.int32(0)))],
    out_specs=pl.BlockSpec((BLK_E, D), lambda i: (i, jnp.int32(0))),
    out_shape=jax.ShapeDtypeStruct((N_EDGES, D), jnp.bfloat16),
)


def _tc1_body(p_ref, w_ref, o_ref):
    nm = p_ref[0] + p_ref[1]
    g = jnp.dot(nm, w_ref[...], preferred_element_type=jnp.float32,
                precision=lax.Precision.HIGHEST)
    col = lax.broadcasted_iota(jnp.int32, g.shape, 1)
    o_ref[...] = jnp.where(col < GATE, jax.nn.sigmoid(g), jnp.tanh(g))


_tc1 = pl.pallas_call(
    _tc1_body,
    grid=(NPAD // BLK_N,),
    in_specs=[pl.BlockSpec((NC, BLK_N, D), lambda i: (jnp.int32(0), i, jnp.int32(0))),
              pl.BlockSpec((D, D), lambda i: (jnp.int32(0), jnp.int32(0)))],
    out_specs=pl.BlockSpec((BLK_N, D), lambda i: (i, jnp.int32(0))),
    out_shape=jax.ShapeDtypeStruct((NPAD, D), jnp.float32),
)


def _tc2_body(s_ref, c_ref, w_ref, o_ref):
    s = s_ref[0] + s_ref[1]
    cnt = c_ref[0, :, 0:1] + c_ref[1, :, 0:1]
    h = s / jnp.maximum(cnt, 1.0)
    o_ref[...] = jnp.dot(h * h, w_ref[...],
                         preferred_element_type=jnp.float32,
                         precision=lax.Precision.HIGHEST)


_tc2 = pl.pallas_call(
    _tc2_body,
    grid=(CPAD // BLK_C,),
    in_specs=[pl.BlockSpec((NC, BLK_C, D), lambda i: (jnp.int32(0), i, jnp.int32(0))),
              pl.BlockSpec((NC, BLK_C, D), lambda i: (jnp.int32(0), i, jnp.int32(0))),
              pl.BlockSpec((D, D), lambda i: (jnp.int32(0), jnp.int32(0)))],
    out_specs=pl.BlockSpec((BLK_C, D), lambda i: (i, jnp.int32(0))),
    out_shape=jax.ShapeDtypeStruct((CPAD, D), jnp.float32),
)


def _tc3_body(g_ref, xa_ref, z_ref, o_ref):
    m = z_ref[...] > 1
    o_ref[...] = jnp.where(m, g_ref[...], xa_ref[...])


_tc3 = pl.pallas_call(
    _tc3_body,
    grid=(NPAD // BLK_N,),
    in_specs=[pl.BlockSpec((BLK_N, D), lambda i: (i, jnp.int32(0))),
              pl.BlockSpec((BLK_N, D), lambda i: (i, jnp.int32(0))),
              pl.BlockSpec((BLK_N, 1), lambda i: (i, jnp.int32(0)))],
    out_specs=pl.BlockSpec((BLK_N, D), lambda i: (i, jnp.int32(0))),
    out_shape=jax.ShapeDtypeStruct((NPAD, D), jnp.float32),
)


def kernel(x, edge_index, edge_attr, z, canonical, W_tp, W_lin, W_heavy):
    x = x.astype(jnp.float32)
    src_p = edge_index[0].astype(jnp.int32).reshape(NW, KE, EC)
    dst_p = edge_index[1].astype(jnp.int32).reshape(NW, KE, EC)

    n_pad_n = NPAD - x.shape[0]
    z_p = jnp.concatenate(
        [z.astype(jnp.int32), jnp.zeros((n_pad_n,), jnp.int32)]).reshape(NW, K2, C2)
    can_p = jnp.concatenate(
        [canonical.astype(jnp.int32), jnp.zeros((n_pad_n,), jnp.int32)]
    ).reshape(NW, K2, C2)

    blk = jnp.stack([jnp.arange(L), L + jnp.arange(L)], axis=1).reshape(2 * L)
    perm = (blk[None, :] + 2 * L * jnp.arange(D // (2 * L))[:, None]).reshape(D)
    wt_perm = W_tp.astype(jnp.float32).T[:, perm]
    a = _tc0(edge_attr.astype(jnp.float32), wt_perm)
    a_i32 = jax.lax.bitcast_convert_type(
        a.reshape(N_EDGES, D // 2, 2), jnp.int32)
    a4 = a_i32.reshape(NW, KE, EC, D // 2)
    partials = _sc1(x, src_p, dst_p, a4)
    x_aggr = _tc1(partials, W_lin.astype(jnp.float32))
    sums = _sc2(x_aggr, z_p, can_p)
    cnts = _sc2(jnp.ones((NPAD, D), jnp.float32), z_p, can_p)
    t = _tc2(sums, cnts, W_heavy.astype(jnp.float32))
    g = _sc3(t, can_p)
    out_p = _tc3(g, x_aggr, z_p.reshape(NPAD, 1))
    return out_p[:x.shape[0]].astype(jnp.float64)


# speedup vs baseline: 1.8504x; 1.8504x over previous
import functools

import jax
import jax.numpy as jnp
from jax import lax
from jax.experimental import pallas as pl
from jax.experimental.pallas import tpu as pltpu
from jax.experimental.pallas import tpu_sc as plsc

N_NODES = 10000
D = 128
D_EDGE = 16
GATE = 16
NUM_CANON = 5000

NC = 2
NS = 16
L = 16
NW = NC * NS

N_EDGES = 320000
EC = 80
KE = 125
E_PER_W = EC * KE
KB = 5
NG = KE // KB

NPAD = 10240
ROWS_PER_TILE = NPAD // NS
ROWS_PER_W = NPAD // NW
CPAD = 5120
CROWS_PER_TILE = CPAD // NS
C2 = 64
K2 = ROWS_PER_W // C2

_MESH = plsc.VectorSubcoreMesh(
    core_axis_name="c", subcore_axis_name="s", num_cores=NC, num_subcores=NS)


KB8 = 8
NB = KE // KB8
KTAIL = KE - NB * KB8


def _sc1_body(x_hbm, src_hbm, dst_hbm, a_hbm, out_hbm,
              src_v, dst8_v, dst5_v, xg0_v, xg1_v, a_v, acc,
              sem0, sem1):
    cid = lax.axis_index("c")
    sid = lax.axis_index("s")
    wid = sid * NC + cid
    xg = (xg0_v, xg1_v)
    gsems = (sem0, sem1)

    def zrow(r, _):
        for c in range(D // L):
            xg0_v[r, pl.ds(c * L, L)] = jnp.zeros((L,), jnp.float32)
        return jnp.int32(0)
    lax.fori_loop(jnp.int32(0), jnp.int32(EC), zrow, jnp.int32(0))
    r0 = sid * ROWS_PER_TILE
    for b in range(ROWS_PER_TILE // EC):
        pltpu.sync_copy(xg0_v, acc.at[pl.ds(r0 + b * EC, EC)])
    plsc.subcore_barrier()

    pltpu.sync_copy(src_hbm.at[wid], src_v)

    def mul(xg_v):
        def row(r, _):
            for k in range(D // (2 * L)):
                ab = plsc.bitcast(a_v[r, pl.ds(k * L, L)], jnp.bfloat16)
                a0, a1 = plsc.unpack(ab, format=plsc.PackFormat.INTERLEAVED,
                                     preferred_element_type=jnp.float32)
                s0 = pl.ds(k * 2 * L, L)
                s1 = pl.ds(k * 2 * L + L, L)
                xg_v[r, s0] = xg_v[r, s0] * a0
                xg_v[r, s1] = xg_v[r, s1] * a1
            return jnp.int32(0)
        lax.fori_loop(jnp.int32(0), jnp.int32(EC), row, jnp.int32(0))

    pltpu.async_copy(x_hbm.at[src_v.at[jnp.int32(0)]], xg0_v, sem0)

    def step(j, c, dst_ref, ci, fire_next):
        p, q = c % 2, (c + 1) % 2
        if fire_next:
            pltpu.async_copy(x_hbm.at[src_v.at[j + 1]], xg[q], gsems[q])
        pltpu.make_async_copy(x_hbm.at[src_v.at[j]], xg[p], gsems[p]).wait()
        pltpu.sync_copy(a_hbm.at[wid, j], a_v)
        mul(xg[p])
        xg_v2 = xg[p]
        pltpu.sync_copy(xg_v2, acc.at[dst_ref.at[jnp.int32(ci)]], add=True)

    def block(b, _):
        pltpu.sync_copy(dst_hbm.at[wid, pl.ds(b * KB8, KB8)], dst8_v)
        for c in range(KB8):
            step(b * KB8 + c, c, dst8_v, c, True)
        return jnp.int32(0)
    lax.fori_loop(jnp.int32(0), jnp.int32(NB), block, jnp.int32(0))

    pltpu.sync_copy(dst_hbm.at[wid, pl.ds(NB * KB8, KTAIL)], dst5_v)
    for c in range(KTAIL):
        step(jnp.int32(NB * KB8 + c), c, dst5_v, c, c + 1 < KTAIL)

    plsc.subcore_barrier()
    for b in range(ROWS_PER_TILE // EC):
        pltpu.sync_copy(acc.at[pl.ds(r0 + b * EC, EC)], xg0_v)
        pltpu.sync_copy(xg0_v, out_hbm.at[cid, pl.ds(r0 + b * EC, EC)])


_sc1 = functools.partial(
    pl.kernel,
    out_type=jax.ShapeDtypeStruct((NC, NPAD, D), jnp.float32),
    mesh=_MESH,
    scratch_types=[
        pltpu.VMEM((KE, EC), jnp.int32),
        pltpu.VMEM((KB8, EC), jnp.int32),
        pltpu.VMEM((KTAIL, EC), jnp.int32),
        pltpu.VMEM((EC, D), jnp.float32),
        pltpu.VMEM((EC, D), jnp.float32),
        pltpu.VMEM((EC, D // 2), jnp.int32),
        pltpu.VMEM_SHARED((NPAD, D), jnp.float32),
        pltpu.SemaphoreType.DMA,
        pltpu.SemaphoreType.DMA,
    ],
    compiler_params=pltpu.CompilerParams(needs_layout_passes=False),
)(_sc1_body)


def _sc2_body(xa_hbm, z_hbm, can_hbm, sums_out, z_v, c_v, seg_v, xa_v, acc):
    cid = lax.axis_index("c")
    sid = lax.axis_index("s")
    wid = sid * NC + cid

    def zrow(r, _):
        for c in range(D // L):
            xa_v[r, pl.ds(c * L, L)] = jnp.zeros((L,), jnp.float32)
        return jnp.int32(0)
    lax.fori_loop(jnp.int32(0), jnp.int32(C2), zrow, jnp.int32(0))
    r0 = sid * CROWS_PER_TILE
    for b in range(CROWS_PER_TILE // C2):
        pltpu.sync_copy(xa_v, acc.at[pl.ds(r0 + b * C2, C2)])
    plsc.subcore_barrier()

    pltpu.sync_copy(z_hbm.at[wid], z_v)
    pltpu.sync_copy(can_hbm.at[wid], c_v)
    base = wid * ROWS_PER_W
    for j in range(K2):
        jj = jnp.int32(j)
        for i in range(C2 // L):
            sl = pl.ds(i * L, L)
            zz = z_v[jj, sl]
            cc = c_v[jj, sl]
            dummy = NUM_CANON + lax.iota(jnp.int32, L) + (i * L)
            seg_v[jj, sl] = jnp.where(zz > 1, cc, dummy)
        pltpu.sync_copy(xa_hbm.at[pl.ds(base + j * C2, C2)], xa_v)
        pltpu.sync_copy(xa_v, acc.at[seg_v.at[jj]], add=True)

    plsc.subcore_barrier()
    for b in range(CROWS_PER_TILE // C2):
        pltpu.sync_copy(acc.at[pl.ds(r0 + b * C2, C2)], xa_v)
        pltpu.sync_copy(xa_v, sums_out.at[cid, pl.ds(r0 + b * C2, C2)])


_sc2 = functools.partial(
    pl.kernel,
    out_type=jax.ShapeDtypeStruct((NC, CPAD, D), jnp.float32),
    mesh=_MESH,
    scratch_types=[
        pltpu.VMEM((K2, C2), jnp.int32),
        pltpu.VMEM((K2, C2), jnp.int32),
        pltpu.VMEM((K2, C2), jnp.int32),
        pltpu.VMEM((C2, D), jnp.float32),
        pltpu.VMEM_SHARED((CPAD, D), jnp.float32),
    ],
)(_sc2_body)


def _sc3_body(t_hbm, can_hbm, out_hbm, c_v, g_v, sem):
    cid = lax.axis_index("c")
    sid = lax.axis_index("s")
    wid = sid * NC + cid
    pltpu.sync_copy(can_hbm.at[wid], c_v)
    base = wid * ROWS_PER_W
    for j in range(K2):
        jj = jnp.int32(j)
        pltpu.async_copy(t_hbm.at[c_v.at[jj]], g_v, sem).wait()
        pltpu.sync_copy(g_v, out_hbm.at[pl.ds(base + j * C2, C2)])


_sc3 = functools.partial(
    pl.kernel,
    out_type=jax.ShapeDtypeStruct((NPAD, D), jnp.float32),
    mesh=_MESH,
    scratch_types=[
        pltpu.VMEM((K2, C2), jnp.int32),
        pltpu.VMEM((C2, D), jnp.float32),
        pltpu.SemaphoreType.DMA,
    ],
)(_sc3_body)


BLK_E = 4000
BLK_N = 1024
BLK_C = 1024


def _tc0_body(ea_ref, wt_lo_ref, wt_hi_ref, o_ref):
    e = ea_ref[...]
    r0 = jnp.dot(e, wt_lo_ref[...], preferred_element_type=jnp.float32,
                 precision=lax.Precision.HIGHEST)
    r1 = jnp.dot(e, wt_hi_ref[...], preferred_element_type=jnp.float32,
                 precision=lax.Precision.HIGHEST)
    b0 = lax.bitcast_convert_type(
        r0.astype(jnp.bfloat16).astype(jnp.float32), jnp.uint32)
    b1 = lax.bitcast_convert_type(
        r1.astype(jnp.bfloat16).astype(jnp.float32), jnp.uint32)
    packed = (b1 & jnp.uint32(0xFFFF0000)) | (b0 >> jnp.uint32(16))
    o_ref[...] = lax.bitcast_convert_type(packed, jnp.int32)


_tc0 = pl.pallas_call(
    _tc0_body,
    grid=(N_EDGES // BLK_E,),
    in_specs=[pl.BlockSpec((BLK_E, D_EDGE), lambda i: (i, jnp.int32(0))),
              pl.BlockSpec((D_EDGE, D // 2), lambda i: (jnp.int32(0), jnp.int32(0))),
              pl.BlockSpec((D_EDGE, D // 2), lambda i: (jnp.int32(0), jnp.int32(0)))],
    out_specs=pl.BlockSpec((BLK_E, D // 2), lambda i: (i, jnp.int32(0))),
    out_shape=jax.ShapeDtypeStruct((N_EDGES, D // 2), jnp.int32),
)


def _tc1_body(p_ref, w_ref, o_ref):
    nm = p_ref[0] + p_ref[1]
    g = jnp.dot(nm, w_ref[...], preferred_element_type=jnp.float32,
                precision=lax.Precision.HIGHEST)
    col = lax.broadcasted_iota(jnp.int32, g.shape, 1)
    o_ref[...] = jnp.where(col < GATE, jax.nn.sigmoid(g), jnp.tanh(g))


_tc1 = pl.pallas_call(
    _tc1_body,
    grid=(NPAD // BLK_N,),
    in_specs=[pl.BlockSpec((NC, BLK_N, D), lambda i: (jnp.int32(0), i, jnp.int32(0))),
              pl.BlockSpec((D, D), lambda i: (jnp.int32(0), jnp.int32(0)))],
    out_specs=pl.BlockSpec((BLK_N, D), lambda i: (i, jnp.int32(0))),
    out_shape=jax.ShapeDtypeStruct((NPAD, D), jnp.float32),
)


def _tc2_body(s_ref, c_ref, w_ref, o_ref):
    s = s_ref[0] + s_ref[1]
    cnt = c_ref[0, :, 0:1] + c_ref[1, :, 0:1]
    h = s / jnp.maximum(cnt, 1.0)
    o_ref[...] = jnp.dot(h * h, w_ref[...],
                         preferred_element_type=jnp.float32,
                         precision=lax.Precision.HIGHEST)


_tc2 = pl.pallas_call(
    _tc2_body,
    grid=(CPAD // BLK_C,),
    in_specs=[pl.BlockSpec((NC, BLK_C, D), lambda i: (jnp.int32(0), i, jnp.int32(0))),
              pl.BlockSpec((NC, BLK_C, D), lambda i: (jnp.int32(0), i, jnp.int32(0))),
              pl.BlockSpec((D, D), lambda i: (jnp.int32(0), jnp.int32(0)))],
    out_specs=pl.BlockSpec((BLK_C, D), lambda i: (i, jnp.int32(0))),
    out_shape=jax.ShapeDtypeStruct((CPAD, D), jnp.float32),
)


def _tc3_body(g_ref, xa_ref, z_ref, o_ref):
    m = z_ref[...] > 1
    o_ref[...] = jnp.where(m, g_ref[...], xa_ref[...])


_tc3 = pl.pallas_call(
    _tc3_body,
    grid=(NPAD // BLK_N,),
    in_specs=[pl.BlockSpec((BLK_N, D), lambda i: (i, jnp.int32(0))),
              pl.BlockSpec((BLK_N, D), lambda i: (i, jnp.int32(0))),
              pl.BlockSpec((BLK_N, 1), lambda i: (i, jnp.int32(0)))],
    out_specs=pl.BlockSpec((BLK_N, D), lambda i: (i, jnp.int32(0))),
    out_shape=jax.ShapeDtypeStruct((NPAD, D), jnp.float32),
)


def kernel(x, edge_index, edge_attr, z, canonical, W_tp, W_lin, W_heavy):
    x = x.astype(jnp.float32)
    src_p = edge_index[0].astype(jnp.int32).reshape(NW, KE, EC)
    dst_p = edge_index[1].astype(jnp.int32).reshape(NW, KE, EC)

    n_pad_n = NPAD - x.shape[0]
    z_p = jnp.concatenate(
        [z.astype(jnp.int32), jnp.zeros((n_pad_n,), jnp.int32)]).reshape(NW, K2, C2)
    can_p = jnp.concatenate(
        [canonical.astype(jnp.int32), jnp.zeros((n_pad_n,), jnp.int32)]
    ).reshape(NW, K2, C2)

    cols_a = (2 * L * jnp.arange(D // (2 * L))[:, None]
              + jnp.arange(L)[None, :]).reshape(D // 2)
    wt = W_tp.astype(jnp.float32).T
    a = _tc0(edge_attr.astype(jnp.float32), wt[:, cols_a], wt[:, cols_a + L])
    a4 = a.reshape(NW, KE, EC, D // 2)
    partials = _sc1(x, src_p, dst_p, a4)
    x_aggr = _tc1(partials, W_lin.astype(jnp.float32))
    sums = _sc2(x_aggr, z_p, can_p)
    cnts = _sc2(jnp.ones((NPAD, D), jnp.float32), z_p, can_p)
    t = _tc2(sums, cnts, W_heavy.astype(jnp.float32))
    g = _sc3(t, can_p)
    out_p = _tc3(g, x_aggr, z_p.reshape(NPAD, 1))
    return out_p[:x.shape[0]].astype(jnp.float64)


# speedup vs baseline: 2.2452x; 1.2134x over previous
import functools

import jax
import jax.numpy as jnp
from jax import lax
from jax.experimental import pallas as pl
from jax.experimental.pallas import tpu as pltpu
from jax.experimental.pallas import tpu_sc as plsc

N_NODES = 10000
D = 128
D_EDGE = 16
GATE = 16
NUM_CANON = 5000

NC = 2
NS = 16
L = 16
NW = NC * NS

N_EDGES = 320000
EC = 80
KE = 125
E_PER_W = EC * KE
KB = 5
NG = KE // KB

NPAD = 10240
ROWS_PER_TILE = NPAD // NS
ROWS_PER_W = NPAD // NW
CPAD = 5120
CROWS_PER_TILE = CPAD // NS
C2 = 64
K2 = ROWS_PER_W // C2

_MESH = plsc.VectorSubcoreMesh(
    core_axis_name="c", subcore_axis_name="s", num_cores=NC, num_subcores=NS)


KB8 = 8
NB = KE // KB8
KTAIL = KE - NB * KB8


def _sc1_body(x_hbm, src_hbm, dst_hbm, a_hbm, out_hbm,
              src_v, dst8_v, dst5_v, xg0_v, xg1_v, a_v, acc,
              sem0, sem1):
    cid = lax.axis_index("c")
    sid = lax.axis_index("s")
    wid = sid * NC + cid
    xg = (xg0_v, xg1_v)
    gsems = (sem0, sem1)

    def zrow(r, _):
        for c in range(D // L):
            xg0_v[r, pl.ds(c * L, L)] = jnp.zeros((L,), jnp.float32)
        return jnp.int32(0)
    lax.fori_loop(jnp.int32(0), jnp.int32(EC), zrow, jnp.int32(0))
    r0 = sid * ROWS_PER_TILE
    for b in range(ROWS_PER_TILE // EC):
        pltpu.sync_copy(xg0_v, acc.at[pl.ds(r0 + b * EC, EC)])
    plsc.subcore_barrier()

    pltpu.sync_copy(src_hbm.at[wid], src_v)

    def mul(xg_v):
        def row(r, _):
            for k in range(D // (2 * L)):
                ab = plsc.bitcast(a_v[r, pl.ds(k * L, L)], jnp.bfloat16)
                a0, a1 = plsc.unpack(ab, format=plsc.PackFormat.INTERLEAVED,
                                     preferred_element_type=jnp.float32)
                s0 = pl.ds(k * 2 * L, L)
                s1 = pl.ds(k * 2 * L + L, L)
                xg_v[r, s0] = xg_v[r, s0] * a0
                xg_v[r, s1] = xg_v[r, s1] * a1
            return jnp.int32(0)
        lax.fori_loop(jnp.int32(0), jnp.int32(EC), row, jnp.int32(0))

    pltpu.async_copy(x_hbm.at[src_v.at[jnp.int32(0)]], xg0_v, sem0)

    def step(j, c, dst_ref, ci, fire_next):
        p, q = c % 2, (c + 1) % 2
        if fire_next:
            pltpu.async_copy(x_hbm.at[src_v.at[j + 1]], xg[q], gsems[q])
        pltpu.make_async_copy(x_hbm.at[src_v.at[j]], xg[p], gsems[p]).wait()
        pltpu.sync_copy(a_hbm.at[wid, j], a_v)
        mul(xg[p])
        xg_v2 = xg[p]
        pltpu.sync_copy(xg_v2, acc.at[dst_ref.at[jnp.int32(ci)]], add=True)

    def block(b, _):
        pltpu.sync_copy(dst_hbm.at[wid, pl.ds(b * KB8, KB8)], dst8_v)
        for c in range(KB8):
            step(b * KB8 + c, c, dst8_v, c, True)
        return jnp.int32(0)
    lax.fori_loop(jnp.int32(0), jnp.int32(NB), block, jnp.int32(0))

    pltpu.sync_copy(dst_hbm.at[wid, pl.ds(NB * KB8, KTAIL)], dst5_v)
    for c in range(KTAIL):
        step(jnp.int32(NB * KB8 + c), c, dst5_v, c, c + 1 < KTAIL)

    plsc.subcore_barrier()
    for b in range(ROWS_PER_TILE // EC):
        pltpu.sync_copy(acc.at[pl.ds(r0 + b * EC, EC)], xg0_v)
        pltpu.sync_copy(xg0_v, out_hbm.at[cid, pl.ds(r0 + b * EC, EC)])


_sc1 = functools.partial(
    pl.kernel,
    out_type=jax.ShapeDtypeStruct((NC, NPAD, D), jnp.float32),
    mesh=_MESH,
    scratch_types=[
        pltpu.VMEM((KE, EC), jnp.int32),
        pltpu.VMEM((KB8, EC), jnp.int32),
        pltpu.VMEM((KTAIL, EC), jnp.int32),
        pltpu.VMEM((EC, D), jnp.float32),
        pltpu.VMEM((EC, D), jnp.float32),
        pltpu.VMEM((EC, D // 2), jnp.int32),
        pltpu.VMEM_SHARED((NPAD, D), jnp.float32),
        pltpu.SemaphoreType.DMA,
        pltpu.SemaphoreType.DMA,
    ],
    compiler_params=pltpu.CompilerParams(needs_layout_passes=False),
)(_sc1_body)


def _sc2_body(xa_hbm, z_hbm, can_hbm, sums_out, z_v, c_v, seg_v, xa_v, acc):
    cid = lax.axis_index("c")
    sid = lax.axis_index("s")
    wid = sid * NC + cid

    def zrow(r, _):
        for c in range(D // L):
            xa_v[r, pl.ds(c * L, L)] = jnp.zeros((L,), jnp.float32)
        return jnp.int32(0)
    lax.fori_loop(jnp.int32(0), jnp.int32(C2), zrow, jnp.int32(0))
    r0 = sid * CROWS_PER_TILE
    for b in range(CROWS_PER_TILE // C2):
        pltpu.sync_copy(xa_v, acc.at[pl.ds(r0 + b * C2, C2)])
    plsc.subcore_barrier()

    pltpu.sync_copy(z_hbm.at[wid], z_v)
    pltpu.sync_copy(can_hbm.at[wid], c_v)
    base = wid * ROWS_PER_W
    for j in range(K2):
        jj = jnp.int32(j)
        for i in range(C2 // L):
            sl = pl.ds(i * L, L)
            zz = z_v[jj, sl]
            cc = c_v[jj, sl]
            dummy = NUM_CANON + lax.iota(jnp.int32, L) + (i * L)
            seg_v[jj, sl] = jnp.where(zz > 1, cc, dummy)
        pltpu.sync_copy(xa_hbm.at[pl.ds(base + j * C2, C2)], xa_v)
        pltpu.sync_copy(xa_v, acc.at[seg_v.at[jj]], add=True)

    plsc.subcore_barrier()
    for b in range(CROWS_PER_TILE // C2):
        pltpu.sync_copy(acc.at[pl.ds(r0 + b * C2, C2)], xa_v)
        pltpu.sync_copy(xa_v, sums_out.at[cid, pl.ds(r0 + b * C2, C2)])


_sc2 = functools.partial(
    pl.kernel,
    out_type=jax.ShapeDtypeStruct((NC, CPAD, D), jnp.float32),
    mesh=_MESH,
    scratch_types=[
        pltpu.VMEM((K2, C2), jnp.int32),
        pltpu.VMEM((K2, C2), jnp.int32),
        pltpu.VMEM((K2, C2), jnp.int32),
        pltpu.VMEM((C2, D), jnp.float32),
        pltpu.VMEM_SHARED((CPAD, D), jnp.float32),
    ],
)(_sc2_body)


def _sc3_body(t_hbm, can_hbm, out_hbm, c_v, g_v, sem):
    cid = lax.axis_index("c")
    sid = lax.axis_index("s")
    wid = sid * NC + cid
    pltpu.sync_copy(can_hbm.at[wid], c_v)
    base = wid * ROWS_PER_W
    for j in range(K2):
        jj = jnp.int32(j)
        pltpu.async_copy(t_hbm.at[c_v.at[jj]], g_v, sem).wait()
        pltpu.sync_copy(g_v, out_hbm.at[pl.ds(base + j * C2, C2)])


_sc3 = functools.partial(
    pl.kernel,
    out_type=jax.ShapeDtypeStruct((NPAD, D), jnp.float32),
    mesh=_MESH,
    scratch_types=[
        pltpu.VMEM((K2, C2), jnp.int32),
        pltpu.VMEM((C2, D), jnp.float32),
        pltpu.SemaphoreType.DMA,
    ],
)(_sc3_body)


BLK_E = 4000
BLK_N = 1024
BLK_C = 1024


def _tc0_body(ea_ref, wt_lo_ref, wt_hi_ref, o_ref):
    e = ea_ref[...]
    r0 = jnp.dot(e, wt_lo_ref[...], preferred_element_type=jnp.float32)
    r1 = jnp.dot(e, wt_hi_ref[...], preferred_element_type=jnp.float32)
    b0 = lax.bitcast_convert_type(
        r0.astype(jnp.bfloat16).astype(jnp.float32), jnp.uint32)
    b1 = lax.bitcast_convert_type(
        r1.astype(jnp.bfloat16).astype(jnp.float32), jnp.uint32)
    packed = (b1 & jnp.uint32(0xFFFF0000)) | (b0 >> jnp.uint32(16))
    o_ref[...] = lax.bitcast_convert_type(packed, jnp.int32)


_tc0 = pl.pallas_call(
    _tc0_body,
    grid=(N_EDGES // BLK_E,),
    in_specs=[pl.BlockSpec((BLK_E, D_EDGE), lambda i: (i, jnp.int32(0))),
              pl.BlockSpec((D_EDGE, D // 2), lambda i: (jnp.int32(0), jnp.int32(0))),
              pl.BlockSpec((D_EDGE, D // 2), lambda i: (jnp.int32(0), jnp.int32(0)))],
    out_specs=pl.BlockSpec((BLK_E, D // 2), lambda i: (i, jnp.int32(0))),
    out_shape=jax.ShapeDtypeStruct((N_EDGES, D // 2), jnp.int32),
)


def _tc1_body(p_ref, w_ref, o_ref):
    nm = p_ref[0] + p_ref[1]
    g = jnp.dot(nm, w_ref[...], preferred_element_type=jnp.float32,
                precision=lax.Precision.HIGHEST)
    col = lax.broadcasted_iota(jnp.int32, g.shape, 1)
    o_ref[...] = jnp.where(col < GATE, jax.nn.sigmoid(g), jnp.tanh(g))


_tc1 = pl.pallas_call(
    _tc1_body,
    grid=(NPAD // BLK_N,),
    in_specs=[pl.BlockSpec((NC, BLK_N, D), lambda i: (jnp.int32(0), i, jnp.int32(0))),
              pl.BlockSpec((D, D), lambda i: (jnp.int32(0), jnp.int32(0)))],
    out_specs=pl.BlockSpec((BLK_N, D), lambda i: (i, jnp.int32(0))),
    out_shape=jax.ShapeDtypeStruct((NPAD, D), jnp.float32),
)


def _tc2_body(s_ref, c_ref, w_ref, o_ref):
    s = s_ref[0] + s_ref[1]
    cnt = c_ref[0, :, 0:1] + c_ref[1, :, 0:1]
    h = s / jnp.maximum(cnt, 1.0)
    o_ref[...] = jnp.dot(h * h, w_ref[...],
                         preferred_element_type=jnp.float32,
                         precision=lax.Precision.HIGHEST)


_tc2 = pl.pallas_call(
    _tc2_body,
    grid=(CPAD // BLK_C,),
    in_specs=[pl.BlockSpec((NC, BLK_C, D), lambda i: (jnp.int32(0), i, jnp.int32(0))),
              pl.BlockSpec((NC, BLK_C, D), lambda i: (jnp.int32(0), i, jnp.int32(0))),
              pl.BlockSpec((D, D), lambda i: (jnp.int32(0), jnp.int32(0)))],
    out_specs=pl.BlockSpec((BLK_C, D), lambda i: (i, jnp.int32(0))),
    out_shape=jax.ShapeDtypeStruct((CPAD, D), jnp.float32),
)


def _tc3_body(g_ref, xa_ref, z_ref, o_ref):
    m = z_ref[...] > 1
    o_ref[...] = jnp.where(m, g_ref[...], xa_ref[...])


_tc3 = pl.pallas_call(
    _tc3_body,
    grid=(NPAD // BLK_N,),
    in_specs=[pl.BlockSpec((BLK_N, D), lambda i: (i, jnp.int32(0))),
              pl.BlockSpec((BLK_N, D), lambda i: (i, jnp.int32(0))),
              pl.BlockSpec((BLK_N, 1), lambda i: (i, jnp.int32(0)))],
    out_specs=pl.BlockSpec((BLK_N, D), lambda i: (i, jnp.int32(0))),
    out_shape=jax.ShapeDtypeStruct((NPAD, D), jnp.float32),
)


def kernel(x, edge_index, edge_attr, z, canonical, W_tp, W_lin, W_heavy):
    x = x.astype(jnp.float32)
    src_p = edge_index[0].astype(jnp.int32).reshape(NW, KE, EC)
    dst_p = edge_index[1].astype(jnp.int32).reshape(NW, KE, EC)

    n_pad_n = NPAD - x.shape[0]
    z_p = jnp.concatenate(
        [z.astype(jnp.int32), jnp.zeros((n_pad_n,), jnp.int32)]).reshape(NW, K2, C2)
    can_p = jnp.concatenate(
        [canonical.astype(jnp.int32), jnp.zeros((n_pad_n,), jnp.int32)]
    ).reshape(NW, K2, C2)

    cols_a = (2 * L * jnp.arange(D // (2 * L))[:, None]
              + jnp.arange(L)[None, :]).reshape(D // 2)
    wt = W_tp.astype(jnp.float32).T
    a = _tc0(edge_attr.astype(jnp.float32), wt[:, cols_a], wt[:, cols_a + L])
    a4 = a.reshape(NW, KE, EC, D // 2)
    partials = _sc1(x, src_p, dst_p, a4)
    x_aggr = _tc1(partials, W_lin.astype(jnp.float32))
    sums = _sc2(x_aggr, z_p, can_p)
    cnts = _sc2(jnp.ones((NPAD, D), jnp.float32), z_p, can_p)
    t = _tc2(sums, cnts, W_heavy.astype(jnp.float32))
    g = _sc3(t, can_p)
    out_p = _tc3(g, x_aggr, z_p.reshape(NPAD, 1))
    return out_p[:x.shape[0]].astype(jnp.float64)
